# baseline (speedup 1.0000x reference)
import functools

import jax
import jax.numpy as jnp
from jax import lax
from jax.experimental import pallas as pl
from jax.experimental.pallas import tpu as pltpu
from jax.experimental.pallas import tpu_sc as plsc

_LANE = 16


@functools.lru_cache(maxsize=None)
def _make_gather(b, h, d, n_workers, chunk_b):
    rows_per_w = b // n_workers
    nchunk = rows_per_w // chunk_b
    assert nchunk * chunk_b * n_workers == b and nchunk % 2 == 0
    assert d % 8 == 0
    dt = d // 8
    bt = b // 128
    mesh = plsc.VectorSubcoreMesh(core_axis_name="c", subcore_axis_name="s")

    @functools.partial(
        pl.kernel,
        mesh=mesh,
        out_type=jax.ShapeDtypeStruct((h, dt, bt, 8, 128), jnp.float32),
        compiler_params=pltpu.CompilerParams(use_tc_tiling_on_sc=False,
                                             needs_layout_passes=False),
        scratch_types=[
            pltpu.VMEM((2, chunk_b, h), jnp.int32),
            pltpu.VMEM((2, chunk_b, h, d), jnp.float32),
            pltpu.VMEM((2, h, dt, 8, chunk_b + 1), jnp.float32),
            pltpu.SemaphoreType.DMA,
            pltpu.SemaphoreType.DMA,
            pltpu.SemaphoreType.DMA,
            pltpu.SemaphoreType.DMA,
            pltpu.SemaphoreType.DMA,
            pltpu.SemaphoreType.DMA,
        ],
    )
    def gather_kernel(idx_hbm, tab_hbm, out_hbm, idx_v, rows_v, t_v,
                      i_sem0, i_sem1, g_sem0, g_sem1, o_sem0, o_sem1):
        i_sems = (i_sem0, i_sem1)
        g_sems = (g_sem0, g_sem1)
        o_sems = (o_sem0, o_sem1)
        wid = lax.axis_index("s") * 2 + lax.axis_index("c")
        base = wid * rows_per_w

        def idx_copy(g, s):
            return pltpu.make_async_copy(
                idx_hbm.at[pl.ds(base + g * chunk_b, chunk_b)],
                idx_v.at[s], i_sems[s])

        def gat_copy(s, j):
            return pltpu.make_async_copy(
                tab_hbm.at[idx_v.at[s, j]],
                rows_v.at[s, j], g_sems[s])

        def out_copy(g, s):
            row0 = base + g * chunk_b
            return pltpu.make_async_copy(
                t_v.at[s, :, :, :, pl.ds(0, chunk_b)],
                out_hbm.at[:, :, row0 // 128,
                           :, pl.ds(row0 % 128, chunk_b)],
                o_sems[s])

        lanes_i = lax.iota(jnp.int32, _LANE)
        s_vecs = tuple(jnp.full((_LANE,), s, jnp.int32) for s in (0, 1))
        dhi_vecs = tuple((lanes_i + (_LANE * hf)) // 8 for hf in range(d // _LANE))
        dlo_vecs = tuple((lanes_i + (_LANE * hf)) % 8 for hf in range(d // _LANE))

        def transpose_chunk(s, hh):
            hh_vec = jnp.full((_LANE,), hh, jnp.int32)
            for b in range(chunk_b):
                b_vec = jnp.full((_LANE,), b, jnp.int32)
                for hf in range(d // _LANE):
                    vals = rows_v[s, b, hh, pl.ds(_LANE * hf, _LANE)]
                    plsc.store_scatter(
                        t_v, [s_vecs[s], hh_vec, dhi_vecs[hf],
                              dlo_vecs[hf], b_vec], vals)

        idx_copy(0, 0).start()
        idx_copy(1, 1).start()
        idx_copy(0, 0).wait()
        for j in range(chunk_b):
            gat_copy(0, j).start()

        def step(i, _):
            for s in (0, 1):
                g = 2 * i + s
                for j in range(chunk_b):
                    gat_copy(s, j).wait()

                @pl.when(g + 1 < nchunk)
                def _():
                    idx_copy(g + 1, 1 - s).wait()
                    for j in range(chunk_b):
                        gat_copy(1 - s, j).start()

                @pl.when(g >= 2)
                def _():
                    out_copy(g - 2, s).wait()

                def tbody(hh, _c):
                    transpose_chunk(s, 2 * hh)
                    transpose_chunk(s, 2 * hh + 1)
                    return _c
                lax.fori_loop(0, h // 2, tbody, None)
                if h % 2:
                    transpose_chunk(s, h - 1)

                out_copy(g, s).start()

                @pl.when(g + 2 < nchunk)
                def _():
                    idx_copy(g + 2, s).start()
            return _

        lax.fori_loop(0, nchunk // 2, step, None)
        out_copy(nchunk - 2, 0).wait()
        out_copy(nchunk - 1, 1).wait()

    return gather_kernel


def kernel(indices, embeddings):
    b, h = indices.shape
    v, d = embeddings.shape
    info = plsc.get_sparse_core_info()
    n_workers = info.num_cores * info.num_subcores
    out_t = _make_gather(b, h, d, n_workers, 16)(indices.astype(jnp.int32),
                                                 embeddings)
    out = out_t.transpose(0, 1, 3, 2, 4).reshape(h, d, b)
    return out.transpose(2, 0, 1)


# speedup vs baseline: 1.0057x; 1.0057x over previous
import functools

import jax
import jax.numpy as jnp
from jax import lax
from jax.experimental import pallas as pl
from jax.experimental.pallas import tpu as pltpu
from jax.experimental.pallas import tpu_sc as plsc

_LANE = 16


@functools.lru_cache(maxsize=None)
def _make_gather(b, h, d, n_workers, chunk_b):
    rows_per_w = b // n_workers
    nchunk = rows_per_w // chunk_b
    assert nchunk * chunk_b * n_workers == b and nchunk % 2 == 0
    assert d % 8 == 0
    dt = d // 8
    bt = b // 128
    mesh = plsc.VectorSubcoreMesh(core_axis_name="c", subcore_axis_name="s")

    @functools.partial(
        pl.kernel,
        mesh=mesh,
        out_type=jax.ShapeDtypeStruct((h, dt, bt, 8, 128), jnp.float32),
        compiler_params=pltpu.CompilerParams(use_tc_tiling_on_sc=False,
                                             needs_layout_passes=False),
        scratch_types=[
            pltpu.VMEM((2, chunk_b, h), jnp.int32),
            pltpu.VMEM((2, chunk_b, h, d), jnp.float32),
            pltpu.VMEM((2, h, d, chunk_b + 1), jnp.float32),
            pltpu.SemaphoreType.DMA,
            pltpu.SemaphoreType.DMA,
            pltpu.SemaphoreType.DMA,
            pltpu.SemaphoreType.DMA,
            pltpu.SemaphoreType.DMA,
            pltpu.SemaphoreType.DMA,
        ],
    )
    def gather_kernel(idx_hbm, tab_hbm, out_hbm, idx_v, rows_v, t_v,
                      i_sem0, i_sem1, g_sem0, g_sem1, o_sem0, o_sem1):
        i_sems = (i_sem0, i_sem1)
        g_sems = (g_sem0, g_sem1)
        o_sems = (o_sem0, o_sem1)
        wid = lax.axis_index("s") * 2 + lax.axis_index("c")
        base = wid * rows_per_w

        def idx_copy(g, s):
            return pltpu.make_async_copy(
                idx_hbm.at[pl.ds(base + g * chunk_b, chunk_b)],
                idx_v.at[s], i_sems[s])

        def gat_copy(s, j):
            return pltpu.make_async_copy(
                tab_hbm.at[idx_v.at[s, j]],
                rows_v.at[s, j], g_sems[s])

        def out_copy(g, s, k):
            row0 = base + g * chunk_b
            return pltpu.make_async_copy(
                t_v.at[s, :, pl.ds(8 * k, 8), pl.ds(0, chunk_b)],
                out_hbm.at[:, k, row0 // 128,
                           :, pl.ds(row0 % 128, chunk_b)],
                o_sems[s])

        lanes_i = lax.iota(jnp.int32, _LANE)
        s_vecs = tuple(jnp.full((_LANE,), s, jnp.int32) for s in (0, 1))
        d_vecs = tuple(lanes_i + (_LANE * hf) for hf in range(d // _LANE))

        def transpose_chunk(s, hh):
            hh_vec = jnp.full((_LANE,), hh, jnp.int32)
            for b in range(chunk_b):
                b_vec = jnp.full((_LANE,), b, jnp.int32)
                for hf in range(d // _LANE):
                    vals = rows_v[s, b, hh, pl.ds(_LANE * hf, _LANE)]
                    plsc.store_scatter(
                        t_v, [s_vecs[s], hh_vec, d_vecs[hf], b_vec], vals)

        idx_copy(0, 0).start()
        idx_copy(1, 1).start()
        idx_copy(0, 0).wait()
        for j in range(chunk_b):
            gat_copy(0, j).start()

        def step(i, _):
            for s in (0, 1):
                g = 2 * i + s
                for j in range(chunk_b):
                    gat_copy(s, j).wait()

                @pl.when(g + 1 < nchunk)
                def _():
                    idx_copy(g + 1, 1 - s).wait()
                    for j in range(chunk_b):
                        gat_copy(1 - s, j).start()

                @pl.when(g >= 2)
                def _():
                    for k in range(dt):
                        out_copy(g - 2, s, k).wait()

                def tbody(hh, _c):
                    transpose_chunk(s, hh)
                    return _c
                lax.fori_loop(0, h, tbody, None)

                for k in range(dt):
                    out_copy(g, s, k).start()

                @pl.when(g + 2 < nchunk)
                def _():
                    idx_copy(g + 2, s).start()
            return _

        lax.fori_loop(0, nchunk // 2, step, None)
        for k in range(dt):
            out_copy(nchunk - 2, 0, k).wait()
            out_copy(nchunk - 1, 1, k).wait()

    return gather_kernel


def kernel(indices, embeddings):
    b, h = indices.shape
    v, d = embeddings.shape
    info = plsc.get_sparse_core_info()
    n_workers = info.num_cores * info.num_subcores
    out_t = _make_gather(b, h, d, n_workers, 16)(indices.astype(jnp.int32),
                                                 embeddings)
    out = out_t.transpose(0, 1, 3, 2, 4).reshape(h, d, b)
    return out.transpose(2, 0, 1)
